# Initial kernel scaffold; baseline (speedup 1.0000x reference)
import jax
import jax.numpy as jnp
from jax.experimental import pallas as pl


def kernel(x, edge_index, W1, b1, W2, b2):
    raise NotImplementedError("write your pallas kernel here")


# speedup vs baseline: 13.9673x; 13.9673x over previous
import functools

import jax
import jax.numpy as jnp
from jax import lax
from jax.experimental import pallas as pl
from jax.experimental.pallas import tpu as pltpu
from jax.experimental.pallas import tpu_sc as plsc

N = 10000
E = 320000
IN_DIM = 128
HID = 256
OUT = 128

NC = 2
NS = 16
CHUNK = 128
N_PAD = 10240
E_PAD = 323584
PAD_IDX = N
RPT = N_PAD // NS
ECHUNKS = E_PAD // CHUNK

_MESH = plsc.VectorSubcoreMesh(core_axis_name="c", subcore_axis_name="s")
_SC_PARAMS = pltpu.CompilerParams(use_tc_tiling_on_sc=False)


@functools.partial(
    pl.kernel,
    out_type=jax.ShapeDtypeStruct((NC, N_PAD, 16), jnp.float32),
    mesh=_MESH,
    compiler_params=_SC_PARAMS,
    scratch_types=[
        pltpu.VMEM((ECHUNKS // (NC * NS), CHUNK), jnp.int32),
        pltpu.VMEM((CHUNK, 16), jnp.float32),
        pltpu.VMEM_SHARED((N_PAD, 16), jnp.float32),
    ],
)
def _deg_sc(dst_hbm, ones_hbm, zeros16_hbm, deg_out, dst_v, ones_v, deg_sh):
    cid = lax.axis_index("c")
    sid = lax.axis_index("s")
    wid = cid * NS + sid
    nchunks = ECHUNKS // (NC * NS)
    pltpu.sync_copy(zeros16_hbm, deg_sh.at[pl.ds(sid * RPT, RPT)])
    pltpu.sync_copy(ones_hbm, ones_v)
    pltpu.sync_copy(dst_hbm.at[wid], dst_v)
    plsc.subcore_barrier()

    def body(c, carry):
        pltpu.sync_copy(ones_v, deg_sh.at[dst_v.at[c]], add=True)
        return carry

    lax.fori_loop(0, nchunks, body, None)
    plsc.subcore_barrier()
    pltpu.sync_copy(deg_sh.at[pl.ds(sid * RPT, RPT)],
                    deg_out.at[cid, pl.ds(sid * RPT, RPT)])


_B2 = 1024


def _lin1_body(x_ref, w1_ref, degp_ref, g1_ref, disw_ref):
    deg = degp_ref[0, :, 0:1] + degp_ref[1, :, 0:1] + 1.0
    dis = lax.rsqrt(deg)
    h = jnp.dot(x_ref[...], w1_ref[...], preferred_element_type=jnp.float32)
    g = h * dis
    g1_ref[0] = g[:, :IN_DIM]
    g1_ref[1] = g[:, IN_DIM:]
    disw_ref[...] = jnp.broadcast_to(dis, (_B2, 16))


def _lin1(x_p, W1, degp):
    return pl.pallas_call(
        _lin1_body,
        grid=(N_PAD // _B2,),
        in_specs=[
            pl.BlockSpec((_B2, IN_DIM), lambda i: (i, 0)),
            pl.BlockSpec((IN_DIM, HID), lambda i: (0, 0)),
            pl.BlockSpec((NC, _B2, 16), lambda i: (0, i, 0)),
        ],
        out_specs=[
            pl.BlockSpec((NC, _B2, IN_DIM), lambda i: (0, i, 0)),
            pl.BlockSpec((_B2, 16), lambda i: (i, 0)),
        ],
        out_shape=[
            jax.ShapeDtypeStruct((NC, N_PAD, IN_DIM), jnp.float32),
            jax.ShapeDtypeStruct((N_PAD, 16), jnp.float32),
        ],
    )(x_p, W1, degp)


_CPT = ECHUNKS // NS


@functools.partial(
    pl.kernel,
    out_type=[
        jax.ShapeDtypeStruct((NC, N_PAD, IN_DIM), jnp.float32),
        jax.ShapeDtypeStruct((NC, N_PAD, 16), jnp.float32),
    ],
    mesh=_MESH,
    compiler_params=_SC_PARAMS,
    scratch_types=[
        pltpu.VMEM((CHUNK,), jnp.int32),
        pltpu.VMEM((CHUNK,), jnp.int32),
        pltpu.VMEM((CHUNK,), jnp.int32),
        pltpu.VMEM((CHUNK, IN_DIM), jnp.float32),
        pltpu.VMEM((CHUNK, 16), jnp.float32),
        pltpu.SemaphoreType.DMA,
        pltpu.SemaphoreType.DMA,
        pltpu.VMEM_SHARED((N_PAD, IN_DIM), jnp.float32),
        pltpu.VMEM_SHARED((N_PAD, 16), jnp.float32),
    ],
)
def _msg_sc(g1flat_hbm, disw_hbm, src_hbm, dst_hbm,
            zeros128_hbm, zeros16_hbm, msg_out, w_out,
            srcw_v, src_v, dst_v, rows_v, wrow_v, sem, wsem, acc_sh, wacc_sh):
    cid = lax.axis_index("c")
    sid = lax.axis_index("s")
    goff = cid * N_PAD
    base = sid * (E_PAD // NS)

    pltpu.sync_copy(zeros128_hbm, acc_sh.at[pl.ds(sid * RPT, RPT)])
    pltpu.sync_copy(zeros16_hbm, wacc_sh.at[pl.ds(sid * RPT, RPT)])
    plsc.subcore_barrier()

    def body(c, carry):
        off = pl.multiple_of(base + c * CHUNK, CHUNK)
        pltpu.sync_copy(src_hbm.at[pl.ds(off, CHUNK)], srcw_v)
        pltpu.sync_copy(dst_hbm.at[pl.ds(off, CHUNK)], dst_v)
        for j in range(CHUNK // 16):
            s = pl.ds(j * 16, 16)
            src_v[s] = srcw_v[s] + goff

        pltpu.async_copy(g1flat_hbm.at[src_v], rows_v, sem).wait()
        pltpu.sync_copy(rows_v, acc_sh.at[dst_v], add=True)

        @pl.when(lax.rem(c, 2) == cid)
        def _():
            pltpu.async_copy(disw_hbm.at[dst_v], wrow_v, wsem).wait()
            pltpu.sync_copy(wrow_v, wacc_sh.at[srcw_v], add=True)

        return carry

    lax.fori_loop(0, _CPT, body, None)
    plsc.subcore_barrier()
    pltpu.sync_copy(acc_sh.at[pl.ds(sid * RPT, RPT)],
                    msg_out.at[cid, pl.ds(sid * RPT, RPT)])
    pltpu.sync_copy(wacc_sh.at[pl.ds(sid * RPT, RPT)],
                    w_out.at[cid, pl.ds(sid * RPT, RPT)])


_B4 = 1000


def _out_body(msg_ref, g1p_ref, disw_ref, wp_ref, b1_ref, w2_ref, b2_ref,
              out_ref):
    i = pl.program_id(0)
    dis = disw_ref[:, 0:1]
    wreal = wp_ref[0, :, 0:1] + wp_ref[1, :, 0:1]
    r0 = jnp.maximum((msg_ref[0] + g1p_ref[0]) * dis + b1_ref[0:1, :], 0.0)
    r1 = jnp.maximum((msg_ref[1] + g1p_ref[1]) * dis + b1_ref[1:2, :], 0.0)
    h2 = (jnp.dot(r0, w2_ref[0], preferred_element_type=jnp.float32)
          + jnp.dot(r1, w2_ref[1], preferred_element_type=jnp.float32))
    wd = (wreal + dis) * dis
    contrib = lax.dot_general(wd, h2, (((0,), (0,)), ((), ())),
                              preferred_element_type=jnp.float32)

    @pl.when(i == 0)
    def _():
        out_ref[...] = jnp.zeros_like(out_ref)

    out_ref[...] += contrib

    @pl.when(i == (N // _B4) - 1)
    def _():
        out_ref[...] = out_ref[...] * (1.0 / N) + b2_ref[...]


def _lin2(msg, g1p, disw, wp, b1h, W2h, b2r):
    return pl.pallas_call(
        _out_body,
        grid=(N // _B4,),
        in_specs=[
            pl.BlockSpec((NC, _B4, IN_DIM), lambda i: (0, i, 0)),
            pl.BlockSpec((NC, _B4, IN_DIM), lambda i: (0, i, 0)),
            pl.BlockSpec((_B4, 16), lambda i: (i, 0)),
            pl.BlockSpec((NC, _B4, 16), lambda i: (0, i, 0)),
            pl.BlockSpec((2, IN_DIM), lambda i: (0, 0)),
            pl.BlockSpec((2, IN_DIM, OUT), lambda i: (0, 0, 0)),
            pl.BlockSpec((1, OUT), lambda i: (0, 0)),
        ],
        out_specs=pl.BlockSpec((1, OUT), lambda i: (0, 0)),
        out_shape=jax.ShapeDtypeStruct((1, OUT), jnp.float32),
    )(msg, g1p, disw, wp, b1h, W2h, b2r)


def kernel(x, edge_index, W1, b1, W2, b2):
    src = edge_index[0].astype(jnp.int32)
    dst = edge_index[1].astype(jnp.int32)
    fill = jnp.full((E_PAD - E,), PAD_IDX, jnp.int32)
    src_p = jnp.concatenate([src, fill])
    dst_p = jnp.concatenate([dst, fill])
    dst_k1 = dst_p.reshape(NC * NS, ECHUNKS // (NC * NS), CHUNK)
    x_p = jnp.pad(x, ((0, N_PAD - N), (0, 0)))
    ones16 = jnp.ones((CHUNK, 16), jnp.float32)
    z16 = jnp.zeros((RPT, 16), jnp.float32)
    z128 = jnp.zeros((RPT, IN_DIM), jnp.float32)

    degp = _deg_sc(dst_k1, ones16, z16)
    g1p, disw = _lin1(x_p, W1, degp)
    g1flat = g1p.reshape(NC * N_PAD, IN_DIM)
    msg, wp = _msg_sc(g1flat, disw, src_p, dst_p, z128, z16)
    out = _lin2(msg, g1p, disw, wp, b1.reshape(2, IN_DIM),
                W2.reshape(2, IN_DIM, OUT), b2.reshape(1, OUT))
    return out.reshape(OUT)
